# baseline (speedup 1.0000x reference)
import functools

import jax
import jax.numpy as jnp
from jax import lax
from jax.experimental import pallas as pl
from jax.experimental.pallas import tpu as pltpu
from jax.experimental.pallas import tpu_sc as plsc

IOU_THRESH = 0.5
MASK_H, MASK_W = 28, 28
N, G, H, W = 5000, 64, 512, 512
NW = 32
NPER = 160
NPAD = 5248
SUB = NPAD // 128
OSZ = MASK_H * MASK_W


def _tc_body(gtb, gtl, props, labels_o, deltas_o, mboxes_o, params_o):
    px1 = props[0]
    py1 = props[1]
    px2 = props[2]
    py2 = props[3]
    area_a = (px2 - px1) * (py2 - py1)

    def step(g, carry):
        biou, bid, blab, bx1, by1, bx2, by2 = carry
        gx1 = gtb[0, g]
        gy1 = gtb[1, g]
        gx2 = gtb[2, g]
        gy2 = gtb[3, g]
        area_b = (gx2 - gx1) * (gy2 - gy1)
        iw = jnp.maximum(jnp.minimum(px2, gx2) - jnp.maximum(px1, gx1), 0.0)
        ih = jnp.maximum(jnp.minimum(py2, gy2) - jnp.maximum(py1, gy1), 0.0)
        inter = iw * ih
        union = jnp.maximum(area_a + area_b - inter, 1e-9)
        iou = inter / union
        take = iou > biou
        return (
            jnp.where(take, iou, biou),
            jnp.where(take, g, bid),
            jnp.where(take, gtl[g], blab),
            jnp.where(take, gx1, bx1),
            jnp.where(take, gy1, by1),
            jnp.where(take, gx2, bx2),
            jnp.where(take, gy2, by2),
        )

    shp = px1.shape
    init = (
        jnp.full(shp, -1.0, jnp.float32),
        jnp.zeros(shp, jnp.int32),
        jnp.zeros(shp, jnp.int32),
        jnp.zeros(shp, jnp.float32),
        jnp.zeros(shp, jnp.float32),
        jnp.zeros(shp, jnp.float32),
        jnp.zeros(shp, jnp.float32),
    )
    biou, bid, blab, bx1, by1, bx2, by2 = lax.fori_loop(0, G, step, init)

    pos = biou >= IOU_THRESH
    labels_o[...] = jnp.where(pos, blab, 0)

    pw = px2 - px1
    ph = py2 - py1
    pcx = px1 + 0.5 * pw
    pcy = py1 + 0.5 * ph
    gw = bx2 - bx1
    gh = by2 - by1
    gcx = bx1 + 0.5 * gw
    gcy = by1 + 0.5 * gh
    zero = jnp.zeros(shp, jnp.float32)
    deltas_o[0] = jnp.where(pos, (gcx - pcx) / pw, zero)
    deltas_o[1] = jnp.where(pos, (gcy - pcy) / ph, zero)
    deltas_o[2] = jnp.where(pos, jnp.log(gw / pw), zero)
    deltas_o[3] = jnp.where(pos, jnp.log(gh / ph), zero)
    mboxes_o[0] = jnp.where(pos, bx1, zero)
    mboxes_o[1] = jnp.where(pos, by1, zero)
    mboxes_o[2] = jnp.where(pos, bx2, zero)
    mboxes_o[3] = jnp.where(pos, by2, zero)

    xi1 = jnp.clip(jnp.round(px1).astype(jnp.int32), 0, W - 1)
    yi1 = jnp.clip(jnp.round(py1).astype(jnp.int32), 0, H - 1)
    xi2 = jnp.clip(jnp.round(px2).astype(jnp.int32), 0, W - 1)
    yi2 = jnp.clip(jnp.round(py2).astype(jnp.int32), 0, H - 1)
    valid = (xi2 > xi1) & (yi2 > yi1)
    gidx = (lax.broadcasted_iota(jnp.int32, shp, 0) * 128
            + lax.broadcasted_iota(jnp.int32, shp, 1))
    flag = (pos & valid & (gidx < N)).astype(jnp.int32)
    izero = jnp.zeros(shp, jnp.int32)
    params_o[0] = bid
    params_o[1] = flag
    params_o[2] = xi1
    params_o[3] = yi1
    params_o[4] = xi2
    params_o[5] = yi2
    params_o[6] = izero
    params_o[7] = izero


_tc_call = pl.pallas_call(
    _tc_body,
    out_shape=[
        jax.ShapeDtypeStruct((SUB, 128), jnp.int32),
        jax.ShapeDtypeStruct((4, SUB, 128), jnp.float32),
        jax.ShapeDtypeStruct((4, SUB, 128), jnp.float32),
        jax.ShapeDtypeStruct((8, SUB, 128), jnp.int32),
    ],
    in_specs=[
        pl.BlockSpec(memory_space=pltpu.SMEM),
        pl.BlockSpec(memory_space=pltpu.SMEM),
        pl.BlockSpec(memory_space=pltpu.VMEM),
    ],
)


def _sc_body(table, params, out, params_v, idx_a, idx_b, buf_a, buf_b,
             out_v, sem_a, sem_b):
    cid = lax.axis_index("c")
    sid = lax.axis_index("s")
    wid = sid * 2 + cid
    base = wid * NPER
    for k in range(6):
        pltpu.sync_copy(params.at[k, pl.ds(base, NPER + 16)],
                        params_v.at[k])

    lanei = lax.iota(jnp.int32, 16)

    def row_grid(i):
        ic = jnp.minimum(i, NPER - 1)
        pv = [params_v[k, pl.ds(ic, 16)][0] for k in range(6)]
        gid, flag = pv[0], pv[1]
        x1i, y1i, x2i, y2i = pv[2], pv[3], pv[4], pv[5]
        y1f = y1i.astype(jnp.float32)
        hc = (y2i - y1i + 1).astype(jnp.float32)
        gbase = gid * H
        rows = []
        wys = []
        for off in (0, 12):
            rs = (lanei + off).astype(jnp.float32)
            ys = jnp.clip((rs + 0.5) * hc / 28.0 - 0.5, 0.0, hc - 1.0) + y1f
            y0 = ys.astype(jnp.int32)
            wys.append(ys - y0.astype(jnp.float32))
            rows.append((gbase + y0, gbase + jnp.minimum(y0 + 1, H - 1)))
        return (gid, flag, x1i, x2i), rows, wys

    def issue(i, idx_v, buf, sem):
        _, rows, _ = row_grid(i)
        for k, off in enumerate((0, 12)):
            idx_v[pl.ds(off, 16)] = rows[k][0]
            idx_v[pl.ds(28 + off, 16)] = rows[k][1]
        pltpu.async_copy(table.at[idx_v], buf, sem)

    def drain(idx_v, buf, sem):
        pltpu.make_async_copy(table.at[idx_v], buf, sem).wait()

    def sample(i, buf):
        (gid, flag, x1i, x2i), _, wys = row_grid(i)

        @pl.when(flag != 0)
        def _():
            x1f = x1i.astype(jnp.float32)
            wc = (x2i - x1i + 1).astype(jnp.float32)
            cols = []
            for ch in range(2):
                cs = (lanei + ch * 16).astype(jnp.float32)
                xs = jnp.clip((cs + 0.5) * wc / 28.0 - 0.5, 0.0, wc - 1.0) + x1f
                x0 = xs.astype(jnp.int32)
                wx = xs - x0.astype(jnp.float32)
                x1n = jnp.minimum(x0 + 1, W - 1)
                cols.append((x0, x1n, wx))

            for r in range(MASK_H):
                wyr = wys[0][r] if r < 12 else wys[1][r - 12]
                row0 = jnp.full((16,), r, jnp.int32)
                row1 = jnp.full((16,), 28 + r, jnp.int32)
                for ch in range(2):
                    x0, x1n, wx = cols[ch]
                    v00 = plsc.load_gather(buf, [row0, x0])
                    v01 = plsc.load_gather(buf, [row0, x1n])
                    v10 = plsc.load_gather(buf, [row1, x0])
                    v11 = plsc.load_gather(buf, [row1, x1n])
                    top = v00 + wx * (v01 - v00)
                    bot = v10 + wx * (v11 - v10)
                    res = top + wyr * (bot - top)
                    out_v[pl.ds(r * MASK_W + ch * 16, 16)] = res

        @pl.when(flag == 0)
        def _():
            zv = jnp.zeros((16,), jnp.float32)
            for j in range(OSZ // 16 + 1):
                out_v[pl.ds(j * 16, 16)] = zv

        @pl.when(base + i < N)
        def _():
            pltpu.sync_copy(out_v.at[pl.ds(0, OSZ)], out.at[base + i])

    issue(0, idx_a, buf_a, sem_a)

    def body(j, carry):
        i0 = 2 * j
        issue(i0 + 1, idx_b, buf_b, sem_b)
        drain(idx_a, buf_a, sem_a)
        sample(i0, buf_a)
        issue(i0 + 2, idx_a, buf_a, sem_a)
        drain(idx_b, buf_b, sem_b)
        sample(i0 + 1, buf_b)
        return carry

    lax.fori_loop(0, NPER // 2, body, 0)
    drain(idx_a, buf_a, sem_a)


_sc_call = pl.kernel(
    _sc_body,
    out_type=jax.ShapeDtypeStruct((N, OSZ), jnp.float32),
    mesh=plsc.VectorSubcoreMesh(core_axis_name="c", subcore_axis_name="s"),
    compiler_params=pltpu.CompilerParams(use_tc_tiling_on_sc=False,
                                         needs_layout_passes=False),
    scratch_types=[
        pltpu.VMEM((6, NPER + 16), jnp.int32),
        pltpu.VMEM((56,), jnp.int32),
        pltpu.VMEM((56,), jnp.int32),
        pltpu.VMEM((56, W), jnp.float32),
        pltpu.VMEM((56, W), jnp.float32),
        pltpu.VMEM((OSZ + 16,), jnp.float32),
        pltpu.SemaphoreType.DMA,
        pltpu.SemaphoreType.DMA,
    ],
)


def kernel(proposals, gt_boxes, gt_labels, gt_masks):
    p = jnp.pad(proposals[0], ((0, NPAD - N), (0, 0)))
    props_pl = p.T.reshape(4, SUB, 128)
    gtb = gt_boxes[0].T
    gtl = gt_labels[0]
    labels_pl, deltas_pl, mboxes_pl, params_pl = _tc_call(gtb, gtl, props_pl)
    labels = labels_pl.reshape(NPAD)[:N][None]
    deltas = deltas_pl.reshape(4, NPAD).T[:N][None]
    mboxes = mboxes_pl.reshape(4, NPAD).T[:N][None]
    params2 = params_pl.reshape(8, NPAD)

    table = gt_masks[0].reshape(G * H, W)
    masks_flat = _sc_call(table, params2)
    masks = masks_flat.reshape(1, N, MASK_H, MASK_W)
    return proposals, labels, deltas, mboxes, masks


# speedup vs baseline: 1.3594x; 1.3594x over previous
import jax
import jax.numpy as jnp
from jax import lax
from jax.experimental import pallas as pl
from jax.experimental.pallas import tpu as pltpu
from jax.experimental.pallas import tpu_sc as plsc

IOU_THRESH = 0.5
MASK_H, MASK_W = 28, 28
N, G, H, W = 5000, 64, 512, 512
NW = 32
NPER = 160
NPAD = 5248
SUB = NPAD // 128
OROW = 32
BLK = 128
NBLK = W // BLK


def _tc_body(gtb, gtl, props, labels_o, deltas_o, mboxes_o, params_o):
    px1 = props[0]
    py1 = props[1]
    px2 = props[2]
    py2 = props[3]
    area_a = (px2 - px1) * (py2 - py1)

    def step(g, carry):
        biou, bid, blab, bx1, by1, bx2, by2 = carry
        gx1 = gtb[0, g]
        gy1 = gtb[1, g]
        gx2 = gtb[2, g]
        gy2 = gtb[3, g]
        area_b = (gx2 - gx1) * (gy2 - gy1)
        iw = jnp.maximum(jnp.minimum(px2, gx2) - jnp.maximum(px1, gx1), 0.0)
        ih = jnp.maximum(jnp.minimum(py2, gy2) - jnp.maximum(py1, gy1), 0.0)
        inter = iw * ih
        union = jnp.maximum(area_a + area_b - inter, 1e-9)
        iou = inter / union
        take = iou > biou
        return (
            jnp.where(take, iou, biou),
            jnp.where(take, g, bid),
            jnp.where(take, gtl[g], blab),
            jnp.where(take, gx1, bx1),
            jnp.where(take, gy1, by1),
            jnp.where(take, gx2, bx2),
            jnp.where(take, gy2, by2),
        )

    shp = px1.shape
    init = (
        jnp.full(shp, -1.0, jnp.float32),
        jnp.zeros(shp, jnp.int32),
        jnp.zeros(shp, jnp.int32),
        jnp.zeros(shp, jnp.float32),
        jnp.zeros(shp, jnp.float32),
        jnp.zeros(shp, jnp.float32),
        jnp.zeros(shp, jnp.float32),
    )
    biou, bid, blab, bx1, by1, bx2, by2 = lax.fori_loop(0, G, step, init)

    pos = biou >= IOU_THRESH
    labels_o[...] = jnp.where(pos, blab, 0)

    pw = px2 - px1
    ph = py2 - py1
    pcx = px1 + 0.5 * pw
    pcy = py1 + 0.5 * ph
    gw = bx2 - bx1
    gh = by2 - by1
    gcx = bx1 + 0.5 * gw
    gcy = by1 + 0.5 * gh
    zero = jnp.zeros(shp, jnp.float32)
    deltas_o[0] = jnp.where(pos, (gcx - pcx) / pw, zero)
    deltas_o[1] = jnp.where(pos, (gcy - pcy) / ph, zero)
    deltas_o[2] = jnp.where(pos, jnp.log(gw / pw), zero)
    deltas_o[3] = jnp.where(pos, jnp.log(gh / ph), zero)
    mboxes_o[0] = jnp.where(pos, bx1, zero)
    mboxes_o[1] = jnp.where(pos, by1, zero)
    mboxes_o[2] = jnp.where(pos, bx2, zero)
    mboxes_o[3] = jnp.where(pos, by2, zero)

    xi1 = jnp.clip(jnp.round(px1).astype(jnp.int32), 0, W - 1)
    yi1 = jnp.clip(jnp.round(py1).astype(jnp.int32), 0, H - 1)
    xi2 = jnp.clip(jnp.round(px2).astype(jnp.int32), 0, W - 1)
    yi2 = jnp.clip(jnp.round(py2).astype(jnp.int32), 0, H - 1)
    valid = (xi2 > xi1) & (yi2 > yi1)
    gidx = (lax.broadcasted_iota(jnp.int32, shp, 0) * 128
            + lax.broadcasted_iota(jnp.int32, shp, 1))
    flag = (pos & valid & (gidx < N)).astype(jnp.int32)
    izero = jnp.zeros(shp, jnp.int32)
    params_o[0] = bid
    params_o[1] = flag
    params_o[2] = xi1
    params_o[3] = yi1
    params_o[4] = xi2
    params_o[5] = yi2
    params_o[6] = izero
    params_o[7] = izero


_tc_call = pl.pallas_call(
    _tc_body,
    out_shape=[
        jax.ShapeDtypeStruct((SUB, 128), jnp.int32),
        jax.ShapeDtypeStruct((4, SUB, 128), jnp.float32),
        jax.ShapeDtypeStruct((4, SUB, 128), jnp.float32),
        jax.ShapeDtypeStruct((8, SUB, 128), jnp.int32),
    ],
    in_specs=[
        pl.BlockSpec(memory_space=pltpu.SMEM),
        pl.BlockSpec(memory_space=pltpu.SMEM),
        pl.BlockSpec(memory_space=pltpu.VMEM),
    ],
)


def _sc_body(table, params, out, params_v, idx1_a, idx2_a, idx1_b, idx2_b,
             buf_a, buf_b, out_v, sem_a, sem_b):
    cid = lax.axis_index("c")
    sid = lax.axis_index("s")
    wid = sid * 2 + cid
    base = wid * NPER
    pltpu.sync_copy(params.at[pl.ds(base, NPER)], params_v)

    lanei = lax.iota(jnp.int32, 16)

    def read_params(i):
        pv = params_v[jnp.minimum(i, NPER - 1), :]
        return pv[0], pv[1], pv[2], pv[3], pv[4], pv[5]

    def xspan(x1i, x2i):
        b0 = x1i >> 7
        narrow = (jnp.minimum(x2i + 1, W - 1) >> 7) - b0 <= 1
        return b0, narrow

    def row_grid(i):
        gid, flag, x1i, y1i, x2i, y2i = read_params(i)
        y1f = y1i.astype(jnp.float32)
        hc = (y2i - y1i + 1).astype(jnp.float32)
        gbase = gid * H
        m0 = []
        m1 = []
        wys = []
        for off in (0, 12):
            rs = (lanei + off).astype(jnp.float32)
            ys = jnp.clip((rs + 0.5) * hc / 28.0 - 0.5, 0.0, hc - 1.0) + y1f
            y0 = ys.astype(jnp.int32)
            wys.append(ys - y0.astype(jnp.float32))
            m0.append((gbase + y0) * NBLK)
            m1.append((gbase + jnp.minimum(y0 + 1, H - 1)) * NBLK)
        return (gid, flag, x1i, x2i), m0, m1, wys

    def issue(i, idx1, idx2, buf, sem):
        (_, _, x1i, x2i), m0, m1, _ = row_grid(i)
        b0, narrow = xspan(x1i, x2i)

        @pl.when(narrow)
        def _():
            b1 = jnp.minimum(b0 + 1, NBLK - 1)
            for k, off in enumerate((0, 12)):
                idx1[pl.ds(off, 16)] = m0[k] + b0
                idx1[pl.ds(28 + off, 16)] = m0[k] + b1
                idx1[pl.ds(56 + off, 16)] = m1[k] + b0
                idx1[pl.ds(84 + off, 16)] = m1[k] + b1
            pltpu.async_copy(table.at[idx1], buf.at[pl.ds(0, 112)], sem)

        @pl.when(jnp.logical_not(narrow))
        def _():
            for k, off in enumerate((0, 12)):
                for j in range(NBLK):
                    idx1[pl.ds(j * 28 + off, 16)] = m0[k] + j
                    idx2[pl.ds(j * 28 + off, 16)] = m1[k] + j
            pltpu.async_copy(table.at[idx1], buf.at[pl.ds(0, 112)], sem)
            pltpu.async_copy(table.at[idx2], buf.at[pl.ds(112, 112)], sem)

    def drain(i, idx1, idx2, buf, sem):
        _, _, x1i, _, x2i, _ = read_params(i)
        _, narrow = xspan(x1i, x2i)
        pltpu.make_async_copy(table.at[idx1], buf.at[pl.ds(0, 112)],
                              sem).wait()

        @pl.when(jnp.logical_not(narrow))
        def _():
            pltpu.make_async_copy(table.at[idx2], buf.at[pl.ds(112, 112)],
                                  sem).wait()

    def sample(i, buf):
        (gid, flag, x1i, x2i), _, _, wys = row_grid(i)
        b0, narrow = xspan(x1i, x2i)
        bb = jnp.where(narrow, b0, 0)
        y1add = jnp.where(narrow, 56, 112)

        @pl.when(flag != 0)
        def _():
            x1f = x1i.astype(jnp.float32)
            wc = (x2i - x1i + 1).astype(jnp.float32)
            cols = []
            for ch in range(2):
                cs = (lanei + ch * 16).astype(jnp.float32)
                xs = jnp.clip((cs + 0.5) * wc / 28.0 - 0.5, 0.0, wc - 1.0) + x1f
                x0 = xs.astype(jnp.int32)
                wx = xs - x0.astype(jnp.float32)
                x1n = jnp.minimum(x0 + 1, W - 1)
                ro0 = ((x0 >> 7) - bb) * 28
                ro1 = ((x1n >> 7) - bb) * 28
                cols.append((x0 & (BLK - 1), ro0, x1n & (BLK - 1), ro1, wx))

            for r in range(MASK_H):
                wyr = wys[0][r] if r < 12 else wys[1][r - 12]
                for ch in range(2):
                    c0, ro0, c1, ro1, wx = cols[ch]
                    r00 = ro0 + r
                    r01 = ro1 + r
                    v00 = plsc.load_gather(buf, [r00, c0])
                    v01 = plsc.load_gather(buf, [r01, c1])
                    v10 = plsc.load_gather(buf, [r00 + y1add, c0])
                    v11 = plsc.load_gather(buf, [r01 + y1add, c1])
                    top = v00 + wx * (v01 - v00)
                    bot = v10 + wx * (v11 - v10)
                    res = top + wyr * (bot - top)
                    out_v[pl.ds(r * OROW + ch * 16, 16)] = res

        @pl.when(flag == 0)
        def _():
            zv = jnp.zeros((16,), jnp.float32)
            for j in range(MASK_H * OROW // 16):
                out_v[pl.ds(j * 16, 16)] = zv

        pltpu.sync_copy(out_v, out.at[base + i])

    issue(0, idx1_a, idx2_a, buf_a, sem_a)

    def body(j, carry):
        i0 = 2 * j
        issue(i0 + 1, idx1_b, idx2_b, buf_b, sem_b)
        drain(i0, idx1_a, idx2_a, buf_a, sem_a)
        sample(i0, buf_a)
        issue(i0 + 2, idx1_a, idx2_a, buf_a, sem_a)
        drain(i0 + 1, idx1_b, idx2_b, buf_b, sem_b)
        sample(i0 + 1, buf_b)
        return carry

    lax.fori_loop(0, NPER // 2, body, 0)
    drain(NPER, idx1_a, idx2_a, buf_a, sem_a)


_sc_call = pl.kernel(
    _sc_body,
    out_type=jax.ShapeDtypeStruct((NW * NPER, MASK_H * OROW), jnp.float32),
    mesh=plsc.VectorSubcoreMesh(core_axis_name="c", subcore_axis_name="s"),
    compiler_params=pltpu.CompilerParams(use_tc_tiling_on_sc=False,
                                         needs_layout_passes=False),
    scratch_types=[
        pltpu.VMEM((NPER, 16), jnp.int32),
        pltpu.VMEM((112,), jnp.int32),
        pltpu.VMEM((112,), jnp.int32),
        pltpu.VMEM((112,), jnp.int32),
        pltpu.VMEM((112,), jnp.int32),
        pltpu.VMEM((224, BLK), jnp.float32),
        pltpu.VMEM((224, BLK), jnp.float32),
        pltpu.VMEM((MASK_H * OROW,), jnp.float32),
        pltpu.SemaphoreType.DMA,
        pltpu.SemaphoreType.DMA,
    ],
)


def kernel(proposals, gt_boxes, gt_labels, gt_masks):
    p = jnp.pad(proposals[0], ((0, NPAD - N), (0, 0)))
    props_pl = p.T.reshape(4, SUB, 128)
    gtb = gt_boxes[0].T
    gtl = gt_labels[0]
    labels_pl, deltas_pl, mboxes_pl, params_pl = _tc_call(gtb, gtl, props_pl)
    labels = labels_pl.reshape(NPAD)[:N][None]
    deltas = deltas_pl.reshape(4, NPAD).T[:N][None]
    mboxes = mboxes_pl.reshape(4, NPAD).T[:N][None]
    params2 = jnp.pad(params_pl.reshape(8, NPAD).T, ((0, 0), (0, 8)))

    table = gt_masks[0].reshape(G * H * NBLK, BLK)
    masks_flat = _sc_call(table, params2)
    masks = masks_flat.reshape(NW * NPER, MASK_H, OROW)[:N, :, :MASK_W][None]
    return proposals, labels, deltas, mboxes, masks
